# Initial kernel scaffold; baseline (speedup 1.0000x reference)
import jax
import jax.numpy as jnp
from jax.experimental import pallas as pl


def kernel(x, frame_indices, pe_weight):
    raise NotImplementedError("write your pallas kernel here")


# speedup vs baseline: 3.4868x; 3.4868x over previous
import functools

import jax
import jax.numpy as jnp
from jax import lax
from jax.experimental import pallas as pl
from jax.experimental.pallas import tpu as pltpu
from jax.experimental.pallas import tpu_sc as plsc

D_MODEL = 128
MAX_FRAMES = 100000

NC, NS, L = 2, 16, 16
NW = NC * NS
CHUNK = 128


def _body(x_hbm, idx_hbm, tbl_hbm, out_hbm, idx_v, rows_v, x_v, sem):
    wid = lax.axis_index("s") * NC + lax.axis_index("c")
    rows_per_w = x_hbm.shape[0] // NW
    n_chunks = rows_per_w // CHUNK
    w_base = wid * rows_per_w

    def chunk_body(j, carry):
        base = w_base + j * CHUNK
        pltpu.sync_copy(idx_hbm.at[pl.ds(base, CHUNK)], idx_v)

        def clamp_body(i, c):
            v = idx_v[pl.ds(i * L, L)]
            idx_v[pl.ds(i * L, L)] = jnp.minimum(
                jnp.maximum(v, 0), MAX_FRAMES - 1)
            return c

        lax.fori_loop(0, CHUNK // L, clamp_body, 0)

        gather = pltpu.async_copy(tbl_hbm.at[idx_v], rows_v, sem)
        pltpu.sync_copy(x_hbm.at[pl.ds(base, CHUNK)], x_v)
        gather.wait()

        def add_body(r, c):
            for col in range(D_MODEL // L):
                s = pl.ds(col * L, L)
                rows_v[r, s] = rows_v[r, s] + x_v[r, s]
            return c

        lax.fori_loop(0, CHUNK, add_body, 0)
        pltpu.sync_copy(rows_v, out_hbm.at[pl.ds(base, CHUNK)])
        return carry

    lax.fori_loop(0, n_chunks, chunk_body, 0)


@functools.partial(jax.jit, static_argnames=())
def kernel(x, frame_indices, pe_weight):
    b, t, d = x.shape
    n_rows = b * t
    x2 = x.reshape(n_rows, d)
    idx = frame_indices.reshape(n_rows).astype(jnp.int32)

    mesh = plsc.VectorSubcoreMesh(
        core_axis_name="c", subcore_axis_name="s",
        num_cores=NC, num_subcores=NS)
    out = pl.kernel(
        _body,
        out_type=jax.ShapeDtypeStruct((n_rows, d), jnp.float32),
        mesh=mesh,
        scratch_types=[
            pltpu.VMEM((CHUNK,), jnp.int32),
            pltpu.VMEM((CHUNK, d), jnp.float32),
            pltpu.VMEM((CHUNK, d), jnp.float32),
            pltpu.SemaphoreType.DMA,
        ],
    )(x2, idx, pe_weight)
    return out.reshape(b, t, d)
